# baseline (speedup 1.0000x reference)
import functools
import numpy as np
import jax
import jax.numpy as jnp
from jax.experimental import pallas as pl
from jax.experimental.pallas import tpu as pltpu
from jax.experimental.pallas import tpu_sc as plsc

S = 2048
H = 768
NH, HD = 12, 64
NR = 14
NS = 2
TOPK = 2
INTER = 768
NRP = 128
RT = 256
AT = 512
TILE = 128
NPAIR = S * TOPK
PBUF = NPAIR + NR * TILE
NBLK = PBUF // TILE
SCALE = 1.0 / np.sqrt(HD)

try:
    _info = plsc.get_sparse_core_info()
    _NC, _NSUB = _info.num_cores, _info.num_subcores
except Exception:
    _NC, _NSUB = 2, 16
NW = _NC * _NSUB


def _gelu(x):
    return 0.5 * x * (1.0 + jax.lax.erf(x * np.float32(1.0 / np.sqrt(2.0))))


def _ln(x, g, b):
    m = jnp.mean(x, axis=-1, keepdims=True)
    v = jnp.mean((x - m) ** 2, axis=-1, keepdims=True)
    return (x - m) * jax.lax.rsqrt(v + 1e-5) * g + b


def _ln_qkv_body(x_ref, g_ref, b_ref, w_ref, bias_ref, o_ref):
    h = _ln(x_ref[...], g_ref[...], b_ref[...])
    o_ref[...] = jnp.dot(h, w_ref[...], preferred_element_type=jnp.float32) + bias_ref[...]


def _attn_body(q_ref, k_ref, v_ref, o_ref):
    q = q_ref[0]
    k = k_ref[0]
    s = jax.lax.dot_general(q, k, (((1,), (1,)), ((), ())),
                            preferred_element_type=jnp.float32) * SCALE
    m = jnp.max(s, axis=-1, keepdims=True)
    e = jnp.exp(s - m)
    sm = jnp.sum(e, axis=-1, keepdims=True)
    ctx = jnp.dot(e, v_ref[0], preferred_element_type=jnp.float32)
    o_ref[0] = ctx * (1.0 / sm)


def _proj_ln_body(c_ref, w_ref, b_ref, x_ref, g_ref, bb_ref, a_ref, h_ref):
    a = jnp.dot(c_ref[...], w_ref[...], preferred_element_type=jnp.float32)
    a = a + b_ref[...] + x_ref[...]
    a_ref[...] = a
    h_ref[...] = _ln(a, g_ref[...], bb_ref[...])


def _shared_router_body(h_ref, a_ref, w1_ref, b1_ref, w2_ref, b2_ref,
                        wr_ref, br_ref, part_ref, tv_ref, ti_ref):
    h = h_ref[...]
    act = _gelu(jnp.dot(h.astype(jnp.bfloat16), w1_ref[...],
                        preferred_element_type=jnp.float32) + b1_ref[...])
    shared = jnp.dot(act.astype(jnp.bfloat16), w2_ref[...],
                     preferred_element_type=jnp.float32) + b2_ref[...]
    part_ref[...] = shared + h + a_ref[...]
    logits = jnp.dot(h, wr_ref[...], preferred_element_type=jnp.float32) + br_ref[...]
    lm = jnp.max(logits, axis=-1, keepdims=True)
    ex = jnp.exp(logits - lm)
    aff = ex / jnp.sum(ex, axis=-1, keepdims=True)
    col = jax.lax.broadcasted_iota(jnp.int32, aff.shape, 1)
    i1 = jnp.argmax(aff, axis=-1)
    m1 = jnp.max(aff, axis=-1)
    masked = jnp.where(col == i1[:, None], -1.0, aff)
    i2 = jnp.argmax(masked, axis=-1)
    m2 = jnp.max(masked, axis=-1)
    tv_ref[...] = jnp.where(col == 0, m1[:, None],
                            jnp.where(col == 1, m2[:, None], 0.0))
    ti_ref[...] = jnp.where(col == 0, i1[:, None].astype(jnp.int32),
                            jnp.where(col == 1, i2[:, None].astype(jnp.int32), 0))


def _moe_ffn_body(se_ref, xg_ref, w1_ref, b1_ref, w2_ref, b2_ref, o_ref):
    del se_ref
    act = _gelu(jnp.dot(xg_ref[...].astype(jnp.bfloat16), w1_ref[0],
                        preferred_element_type=jnp.float32) + b1_ref[0])
    o_ref[...] = jnp.dot(act.astype(jnp.bfloat16), w2_ref[0],
                         preferred_element_type=jnp.float32) + b2_ref[0]


def _combine_body(p_ref, tv_ref, y_ref, o_ref):
    y = y_ref[...]
    tv = tv_ref[...]
    o_ref[...] = (p_ref[...] + tv[:, 0:1] * y[:, :H] + tv[:, 1:2] * y[:, H:])


def _sc_dispatch(h2, dst_even, dst_odd):
    rows_per_w = S // NW
    mesh = plsc.VectorSubcoreMesh(core_axis_name="c", subcore_axis_name="s")

    @functools.partial(
        pl.kernel, mesh=mesh,
        out_type=jax.ShapeDtypeStruct((PBUF, H), jnp.float32),
        scratch_types=[
            pltpu.VMEM((rows_per_w,), jnp.int32),
            pltpu.VMEM((rows_per_w,), jnp.int32),
            pltpu.VMEM((rows_per_w, H), jnp.float32),
            pltpu.SemaphoreType.DMA,
        ],
    )
    def k(h2_hbm, de_hbm, do_hbm, out_hbm, ie_v, io_v, rows_v, sem):
        wid = jax.lax.axis_index("s") * _NC + jax.lax.axis_index("c")
        base = wid * rows_per_w
        pltpu.sync_copy(de_hbm.at[pl.ds(base, rows_per_w)], ie_v)
        pltpu.sync_copy(do_hbm.at[pl.ds(base, rows_per_w)], io_v)
        pltpu.sync_copy(h2_hbm.at[pl.ds(base, rows_per_w)], rows_v)
        pltpu.async_copy(rows_v, out_hbm.at[ie_v], sem).wait()
        pltpu.async_copy(rows_v, out_hbm.at[io_v], sem).wait()

    return k(h2, dst_even, dst_odd)


def _sc_gather(table, idx):
    n = idx.shape[0]
    nblk = n // TILE
    mesh = plsc.VectorSubcoreMesh(core_axis_name="c", subcore_axis_name="s")

    @functools.partial(
        pl.kernel, mesh=mesh,
        out_type=jax.ShapeDtypeStruct((n, H), jnp.float32),
        scratch_types=[
            pltpu.VMEM((TILE,), jnp.int32),
            pltpu.VMEM((TILE, H), jnp.float32),
            pltpu.SemaphoreType.DMA,
        ],
    )
    def k(table_hbm, idx_hbm, out_hbm, idx_v, rows_v, sem):
        wid = jax.lax.axis_index("s") * _NC + jax.lax.axis_index("c")
        for j in range((nblk + NW - 1) // NW):
            t = wid + j * NW

            @pl.when(t < nblk)
            def _do():
                base = t * TILE
                pltpu.sync_copy(idx_hbm.at[pl.ds(base, TILE)], idx_v)
                pltpu.async_copy(table_hbm.at[idx_v], rows_v, sem).wait()
                pltpu.sync_copy(rows_v, out_hbm.at[pl.ds(base, TILE)])

    return k(table, idx)


def _ln_qkv(x2, g, b, wqkv, bqkv):
    return pl.pallas_call(
        _ln_qkv_body,
        grid=(S // RT,),
        in_specs=[
            pl.BlockSpec((RT, H), lambda i: (i, 0)),
            pl.BlockSpec((1, H), lambda i: (0, 0)),
            pl.BlockSpec((1, H), lambda i: (0, 0)),
            pl.BlockSpec((H, 3 * H), lambda i: (0, 0)),
            pl.BlockSpec((1, 3 * H), lambda i: (0, 0)),
        ],
        out_specs=pl.BlockSpec((RT, 3 * H), lambda i: (i, 0)),
        out_shape=jax.ShapeDtypeStruct((S, 3 * H), jnp.float32),
    )(x2, g, b, wqkv, bqkv)


def _attention(q, k, v):
    return pl.pallas_call(
        _attn_body,
        grid=(NH, S // AT),
        in_specs=[
            pl.BlockSpec((1, AT, HD), lambda h, r: (h, r, 0)),
            pl.BlockSpec((1, S, HD), lambda h, r: (h, 0, 0)),
            pl.BlockSpec((1, S, HD), lambda h, r: (h, 0, 0)),
        ],
        out_specs=pl.BlockSpec((1, AT, HD), lambda h, r: (h, r, 0)),
        out_shape=jax.ShapeDtypeStruct((NH, S, HD), jnp.float32),
    )(q, k, v)


def _proj_ln(ctx, wo, bo, x2, g2, b2):
    return pl.pallas_call(
        _proj_ln_body,
        grid=(S // RT,),
        in_specs=[
            pl.BlockSpec((RT, H), lambda i: (i, 0)),
            pl.BlockSpec((H, H), lambda i: (0, 0)),
            pl.BlockSpec((1, H), lambda i: (0, 0)),
            pl.BlockSpec((RT, H), lambda i: (i, 0)),
            pl.BlockSpec((1, H), lambda i: (0, 0)),
            pl.BlockSpec((1, H), lambda i: (0, 0)),
        ],
        out_specs=[
            pl.BlockSpec((RT, H), lambda i: (i, 0)),
            pl.BlockSpec((RT, H), lambda i: (i, 0)),
        ],
        out_shape=[
            jax.ShapeDtypeStruct((S, H), jnp.float32),
            jax.ShapeDtypeStruct((S, H), jnp.float32),
        ],
    )(ctx, wo, bo, x2, g2, b2)


def _shared_router(h2, a, w1c, b1c, w2c, b2s, wr, br):
    return pl.pallas_call(
        _shared_router_body,
        grid=(S // RT,),
        in_specs=[
            pl.BlockSpec((RT, H), lambda i: (i, 0)),
            pl.BlockSpec((RT, H), lambda i: (i, 0)),
            pl.BlockSpec((H, NS * INTER), lambda i: (0, 0)),
            pl.BlockSpec((1, NS * INTER), lambda i: (0, 0)),
            pl.BlockSpec((NS * INTER, H), lambda i: (0, 0)),
            pl.BlockSpec((1, H), lambda i: (0, 0)),
            pl.BlockSpec((H, NRP), lambda i: (0, 0)),
            pl.BlockSpec((1, NRP), lambda i: (0, 0)),
        ],
        out_specs=[
            pl.BlockSpec((RT, H), lambda i: (i, 0)),
            pl.BlockSpec((RT, NRP), lambda i: (i, 0)),
            pl.BlockSpec((RT, NRP), lambda i: (i, 0)),
        ],
        out_shape=[
            jax.ShapeDtypeStruct((S, H), jnp.float32),
            jax.ShapeDtypeStruct((S, NRP), jnp.float32),
            jax.ShapeDtypeStruct((S, NRP), jnp.int32),
        ],
    )(h2, a, w1c, b1c, w2c, b2s, wr, br)


def _moe_ffn(tile_expert, xg, rW1, rb1, rW2, rb2):
    grid_spec = pltpu.PrefetchScalarGridSpec(
        num_scalar_prefetch=1,
        grid=(NBLK,),
        in_specs=[
            pl.BlockSpec((TILE, H), lambda t, se: (t, 0)),
            pl.BlockSpec((1, H, INTER), lambda t, se: (se[t], 0, 0)),
            pl.BlockSpec((1, 1, INTER), lambda t, se: (se[t], 0, 0)),
            pl.BlockSpec((1, INTER, H), lambda t, se: (se[t], 0, 0)),
            pl.BlockSpec((1, 1, H), lambda t, se: (se[t], 0, 0)),
        ],
        out_specs=pl.BlockSpec((TILE, H), lambda t, se: (t, 0)),
    )
    return pl.pallas_call(
        _moe_ffn_body,
        grid_spec=grid_spec,
        out_shape=jax.ShapeDtypeStruct((PBUF, H), jnp.float32),
    )(tile_expert, xg, rW1, rb1[:, None, :], rW2, rb2[:, None, :])


def _combine(partial, tvp, yp2):
    return pl.pallas_call(
        _combine_body,
        grid=(S // RT,),
        in_specs=[
            pl.BlockSpec((RT, H), lambda i: (i, 0)),
            pl.BlockSpec((RT, NRP), lambda i: (i, 0)),
            pl.BlockSpec((RT, 2 * H), lambda i: (i, 0)),
        ],
        out_specs=pl.BlockSpec((RT, H), lambda i: (i, 0)),
        out_shape=jax.ShapeDtypeStruct((S, H), jnp.float32),
    )(partial, tvp, yp2)


def _route_indices(ti):
    e_p = ti.reshape(NPAIR)
    oh = (e_p[:, None] == jnp.arange(NR, dtype=jnp.int32)[None, :]).astype(jnp.int32)
    pc = jnp.cumsum(oh, axis=0)
    rank = jnp.take_along_axis(pc, e_p[:, None], axis=1)[:, 0] - 1
    counts = pc[-1]
    tiles_per = (counts + TILE - 1) // TILE
    ends = jnp.cumsum(tiles_per)
    base = (jnp.concatenate([jnp.zeros((1,), ends.dtype), ends[:-1]]) * TILE).astype(jnp.int32)
    dst = base[e_p] + rank
    tile_expert = jnp.minimum(
        jnp.searchsorted(ends, jnp.arange(NBLK, dtype=ends.dtype), side="right"),
        NR - 1).astype(jnp.int32)
    return dst, tile_expert


def kernel(x, ln1_g, ln1_b, ln2_g, ln2_b, Wq, bq, Wk, bk, Wv, bv, Wo, bo,
           Wr, br, sW1, sb1, sW2, sb2, rW1, rb1, rW2, rb2):
    x2 = x[0]

    wqkv = jnp.concatenate([Wq, Wk, Wv], axis=1)
    bqkv = jnp.concatenate([bq, bk, bv])[None, :]
    qkv = _ln_qkv(x2, ln1_g[None, :], ln1_b[None, :], wqkv, bqkv)

    qkv3 = qkv.reshape(S, 3, NH, HD).transpose(1, 2, 0, 3)
    ctx = _attention(qkv3[0], qkv3[1], qkv3[2])
    ctx2 = ctx.transpose(1, 0, 2).reshape(S, NH * HD)

    a, h2 = _proj_ln(ctx2, Wo, bo[None, :], x2, ln2_g[None, :], ln2_b[None, :])

    w1c = jnp.concatenate([sW1[0], sW1[1]], axis=1).astype(jnp.bfloat16)
    b1c = jnp.concatenate([sb1[0], sb1[1]])[None, :]
    w2c = jnp.concatenate([sW2[0], sW2[1]], axis=0).astype(jnp.bfloat16)
    b2s = (sb2[0] + sb2[1])[None, :]
    wr_p = jnp.zeros((H, NRP), jnp.float32).at[:, :NR].set(Wr)
    br_p = jnp.full((NRP,), -1e30, jnp.float32).at[:NR].set(br)[None, :]
    partial, tvp, tip = _shared_router(h2, a, w1c, b1c, w2c, b2s, wr_p, br_p)

    dst, tile_expert = _route_indices(tip[:, :TOPK])
    dst2 = dst.reshape(S, TOPK)
    xg = _sc_dispatch(h2, dst2[:, 0], dst2[:, 1])
    y_pad = _moe_ffn(tile_expert, xg, rW1.astype(jnp.bfloat16), rb1,
                     rW2.astype(jnp.bfloat16), rb2)
    yp = _sc_gather(y_pad, dst)
    out = _combine(partial, tvp, yp.reshape(S, TOPK * H))
    return out[None]


# speedup vs baseline: 1.3576x; 1.3576x over previous
import functools
import numpy as np
import jax
import jax.numpy as jnp
from jax.experimental import pallas as pl
from jax.experimental.pallas import tpu as pltpu
from jax.experimental.pallas import tpu_sc as plsc

S = 2048
H = 768
NH, HD = 12, 64
NR = 14
NS = 2
TOPK = 2
INTER = 768
NRP = 128
RT = 256
AT = 512
TILE = 128
NPAIR = S * TOPK
PBUF = NPAIR + NR * TILE
NBLK = PBUF // TILE
SCALE = 1.0 / np.sqrt(HD)

try:
    _info = plsc.get_sparse_core_info()
    _NC, _NSUB = _info.num_cores, _info.num_subcores
except Exception:
    _NC, _NSUB = 2, 16
NW = _NC * _NSUB


def _gelu(x):
    return 0.5 * x * (1.0 + jax.lax.erf(x * np.float32(1.0 / np.sqrt(2.0))))


def _ln(x, g, b):
    m = jnp.mean(x, axis=-1, keepdims=True)
    v = jnp.mean((x - m) ** 2, axis=-1, keepdims=True)
    return (x - m) * jax.lax.rsqrt(v + 1e-5) * g + b


def _ln_qkv_body(x_ref, g_ref, b_ref, w_ref, bias_ref, o_ref):
    h = _ln(x_ref[...], g_ref[...], b_ref[...])
    o_ref[...] = jnp.dot(h, w_ref[...], preferred_element_type=jnp.float32) + bias_ref[...]


def _attn_body(q_ref, k_ref, v_ref, o_ref):
    q = q_ref[...].astype(jnp.bfloat16)
    k = k_ref[...].astype(jnp.bfloat16)
    v = v_ref[...].astype(jnp.bfloat16)
    for h in range(NH):
        sl = slice(h * HD, (h + 1) * HD)
        s = jax.lax.dot_general(q[:, sl], k[:, sl], (((1,), (1,)), ((), ())),
                                preferred_element_type=jnp.float32) * SCALE
        m = jnp.max(s, axis=-1, keepdims=True)
        e = jnp.exp(s - m)
        sm = jnp.sum(e, axis=-1, keepdims=True)
        ctx = jnp.dot(e.astype(jnp.bfloat16), v[:, sl],
                      preferred_element_type=jnp.float32)
        o_ref[:, sl] = ctx * (1.0 / sm)


def _proj_ln_body(c_ref, w_ref, b_ref, x_ref, g_ref, bb_ref, a_ref, h_ref):
    a = jnp.dot(c_ref[...], w_ref[...], preferred_element_type=jnp.float32)
    a = a + b_ref[...] + x_ref[...]
    a_ref[...] = a
    h_ref[...] = _ln(a, g_ref[...], bb_ref[...])


def _shared_router_body(h_ref, a_ref, w1_ref, b1_ref, w2_ref, b2_ref,
                        wr_ref, br_ref, part_ref, tv_ref, ti_ref):
    h = h_ref[...]
    act = _gelu(jnp.dot(h.astype(jnp.bfloat16), w1_ref[...],
                        preferred_element_type=jnp.float32) + b1_ref[...])
    shared = jnp.dot(act.astype(jnp.bfloat16), w2_ref[...],
                     preferred_element_type=jnp.float32) + b2_ref[...]
    part_ref[...] = shared + h + a_ref[...]
    logits = jnp.dot(h, wr_ref[...], preferred_element_type=jnp.float32) + br_ref[...]
    lm = jnp.max(logits, axis=-1, keepdims=True)
    ex = jnp.exp(logits - lm)
    aff = ex / jnp.sum(ex, axis=-1, keepdims=True)
    col = jax.lax.broadcasted_iota(jnp.int32, aff.shape, 1)
    i1 = jnp.argmax(aff, axis=-1)
    m1 = jnp.max(aff, axis=-1)
    masked = jnp.where(col == i1[:, None], -1.0, aff)
    i2 = jnp.argmax(masked, axis=-1)
    m2 = jnp.max(masked, axis=-1)
    tv_ref[...] = jnp.where(col == 0, m1[:, None],
                            jnp.where(col == 1, m2[:, None], 0.0))
    ti_ref[...] = jnp.where(col == 0, i1[:, None].astype(jnp.int32),
                            jnp.where(col == 1, i2[:, None].astype(jnp.int32), 0))


def _moe_ffn_body(se_ref, xg_ref, w1_ref, b1_ref, w2_ref, b2_ref, o_ref):
    del se_ref
    act = _gelu(jnp.dot(xg_ref[...].astype(jnp.bfloat16), w1_ref[0],
                        preferred_element_type=jnp.float32) + b1_ref[0])
    o_ref[...] = jnp.dot(act.astype(jnp.bfloat16), w2_ref[0],
                         preferred_element_type=jnp.float32) + b2_ref[0]


def _combine_body(p_ref, tv_ref, y_ref, o_ref):
    y = y_ref[...]
    tv = tv_ref[...]
    o_ref[...] = (p_ref[...] + tv[:, 0:1] * y[:, :H] + tv[:, 1:2] * y[:, H:])


def _sc_dispatch(h2, dst_even, dst_odd):
    rows_per_w = S // NW
    mesh = plsc.VectorSubcoreMesh(core_axis_name="c", subcore_axis_name="s")

    @functools.partial(
        pl.kernel, mesh=mesh,
        out_type=jax.ShapeDtypeStruct((PBUF, H), jnp.float32),
        scratch_types=[
            pltpu.VMEM((rows_per_w,), jnp.int32),
            pltpu.VMEM((rows_per_w,), jnp.int32),
            pltpu.VMEM((rows_per_w, H), jnp.float32),
            pltpu.SemaphoreType.DMA,
        ],
    )
    def k(h2_hbm, de_hbm, do_hbm, out_hbm, ie_v, io_v, rows_v, sem):
        wid = jax.lax.axis_index("s") * _NC + jax.lax.axis_index("c")
        base = wid * rows_per_w
        pltpu.sync_copy(de_hbm.at[pl.ds(base, rows_per_w)], ie_v)
        pltpu.sync_copy(do_hbm.at[pl.ds(base, rows_per_w)], io_v)
        pltpu.sync_copy(h2_hbm.at[pl.ds(base, rows_per_w)], rows_v)
        pltpu.async_copy(rows_v, out_hbm.at[ie_v], sem).wait()
        pltpu.async_copy(rows_v, out_hbm.at[io_v], sem).wait()

    return k(h2, dst_even, dst_odd)


def _sc_gather(table, idx):
    n = idx.shape[0]
    nblk = n // TILE
    mesh = plsc.VectorSubcoreMesh(core_axis_name="c", subcore_axis_name="s")

    @functools.partial(
        pl.kernel, mesh=mesh,
        out_type=jax.ShapeDtypeStruct((n, H), jnp.float32),
        scratch_types=[
            pltpu.VMEM((TILE,), jnp.int32),
            pltpu.VMEM((TILE, H), jnp.float32),
            pltpu.SemaphoreType.DMA,
        ],
    )
    def k(table_hbm, idx_hbm, out_hbm, idx_v, rows_v, sem):
        wid = jax.lax.axis_index("s") * _NC + jax.lax.axis_index("c")
        for j in range((nblk + NW - 1) // NW):
            t = wid + j * NW

            @pl.when(t < nblk)
            def _do():
                base = t * TILE
                pltpu.sync_copy(idx_hbm.at[pl.ds(base, TILE)], idx_v)
                pltpu.async_copy(table_hbm.at[idx_v], rows_v, sem).wait()
                pltpu.sync_copy(rows_v, out_hbm.at[pl.ds(base, TILE)])

    return k(table, idx)


def _ln_qkv(x2, g, b, wqkv, bqkv):
    return pl.pallas_call(
        _ln_qkv_body,
        grid=(S // RT,),
        in_specs=[
            pl.BlockSpec((RT, H), lambda i: (i, 0)),
            pl.BlockSpec((1, H), lambda i: (0, 0)),
            pl.BlockSpec((1, H), lambda i: (0, 0)),
            pl.BlockSpec((H, 3 * H), lambda i: (0, 0)),
            pl.BlockSpec((1, 3 * H), lambda i: (0, 0)),
        ],
        out_specs=pl.BlockSpec((RT, 3 * H), lambda i: (i, 0)),
        out_shape=jax.ShapeDtypeStruct((S, 3 * H), jnp.float32),
    )(x2, g, b, wqkv, bqkv)


def _attention(qkv):
    return pl.pallas_call(
        _attn_body,
        grid=(S // AT,),
        in_specs=[
            pl.BlockSpec((AT, H), lambda r: (r, 0)),
            pl.BlockSpec((S, H), lambda r: (0, 1)),
            pl.BlockSpec((S, H), lambda r: (0, 2)),
        ],
        out_specs=pl.BlockSpec((AT, H), lambda r: (r, 0)),
        out_shape=jax.ShapeDtypeStruct((S, H), jnp.float32),
    )(qkv, qkv, qkv)


def _proj_ln(ctx, wo, bo, x2, g2, b2):
    return pl.pallas_call(
        _proj_ln_body,
        grid=(S // RT,),
        in_specs=[
            pl.BlockSpec((RT, H), lambda i: (i, 0)),
            pl.BlockSpec((H, H), lambda i: (0, 0)),
            pl.BlockSpec((1, H), lambda i: (0, 0)),
            pl.BlockSpec((RT, H), lambda i: (i, 0)),
            pl.BlockSpec((1, H), lambda i: (0, 0)),
            pl.BlockSpec((1, H), lambda i: (0, 0)),
        ],
        out_specs=[
            pl.BlockSpec((RT, H), lambda i: (i, 0)),
            pl.BlockSpec((RT, H), lambda i: (i, 0)),
        ],
        out_shape=[
            jax.ShapeDtypeStruct((S, H), jnp.float32),
            jax.ShapeDtypeStruct((S, H), jnp.float32),
        ],
    )(ctx, wo, bo, x2, g2, b2)


def _shared_router(h2, a, w1c, b1c, w2c, b2s, wr, br):
    return pl.pallas_call(
        _shared_router_body,
        grid=(S // RT,),
        in_specs=[
            pl.BlockSpec((RT, H), lambda i: (i, 0)),
            pl.BlockSpec((RT, H), lambda i: (i, 0)),
            pl.BlockSpec((H, NS * INTER), lambda i: (0, 0)),
            pl.BlockSpec((1, NS * INTER), lambda i: (0, 0)),
            pl.BlockSpec((NS * INTER, H), lambda i: (0, 0)),
            pl.BlockSpec((1, H), lambda i: (0, 0)),
            pl.BlockSpec((H, NRP), lambda i: (0, 0)),
            pl.BlockSpec((1, NRP), lambda i: (0, 0)),
        ],
        out_specs=[
            pl.BlockSpec((RT, H), lambda i: (i, 0)),
            pl.BlockSpec((RT, NRP), lambda i: (i, 0)),
            pl.BlockSpec((RT, NRP), lambda i: (i, 0)),
        ],
        out_shape=[
            jax.ShapeDtypeStruct((S, H), jnp.float32),
            jax.ShapeDtypeStruct((S, NRP), jnp.float32),
            jax.ShapeDtypeStruct((S, NRP), jnp.int32),
        ],
    )(h2, a, w1c, b1c, w2c, b2s, wr, br)


def _moe_ffn(tile_expert, xg, rW1, rb1, rW2, rb2):
    grid_spec = pltpu.PrefetchScalarGridSpec(
        num_scalar_prefetch=1,
        grid=(NBLK,),
        in_specs=[
            pl.BlockSpec((TILE, H), lambda t, se: (t, 0)),
            pl.BlockSpec((1, H, INTER), lambda t, se: (se[t], 0, 0)),
            pl.BlockSpec((1, 1, INTER), lambda t, se: (se[t], 0, 0)),
            pl.BlockSpec((1, INTER, H), lambda t, se: (se[t], 0, 0)),
            pl.BlockSpec((1, 1, H), lambda t, se: (se[t], 0, 0)),
        ],
        out_specs=pl.BlockSpec((TILE, H), lambda t, se: (t, 0)),
    )
    return pl.pallas_call(
        _moe_ffn_body,
        grid_spec=grid_spec,
        out_shape=jax.ShapeDtypeStruct((PBUF, H), jnp.float32),
    )(tile_expert, xg, rW1, rb1[:, None, :], rW2, rb2[:, None, :])


def _combine(partial, tvp, yp2):
    return pl.pallas_call(
        _combine_body,
        grid=(S // RT,),
        in_specs=[
            pl.BlockSpec((RT, H), lambda i: (i, 0)),
            pl.BlockSpec((RT, NRP), lambda i: (i, 0)),
            pl.BlockSpec((RT, 2 * H), lambda i: (i, 0)),
        ],
        out_specs=pl.BlockSpec((RT, H), lambda i: (i, 0)),
        out_shape=jax.ShapeDtypeStruct((S, H), jnp.float32),
    )(partial, tvp, yp2)


def _route_indices(ti):
    e_p = ti.reshape(NPAIR)
    oh = (e_p[:, None] == jnp.arange(NR, dtype=jnp.int32)[None, :]).astype(jnp.int32)
    pc = jnp.cumsum(oh, axis=0)
    rank = jnp.take_along_axis(pc, e_p[:, None], axis=1)[:, 0] - 1
    counts = pc[-1]
    tiles_per = (counts + TILE - 1) // TILE
    ends = jnp.cumsum(tiles_per)
    base = (jnp.concatenate([jnp.zeros((1,), ends.dtype), ends[:-1]]) * TILE).astype(jnp.int32)
    dst = base[e_p] + rank
    tile_expert = jnp.minimum(
        jnp.searchsorted(ends, jnp.arange(NBLK, dtype=ends.dtype), side="right"),
        NR - 1).astype(jnp.int32)
    return dst, tile_expert


def kernel(x, ln1_g, ln1_b, ln2_g, ln2_b, Wq, bq, Wk, bk, Wv, bv, Wo, bo,
           Wr, br, sW1, sb1, sW2, sb2, rW1, rb1, rW2, rb2):
    x2 = x[0]

    wqkv = jnp.concatenate([Wq, Wk, Wv], axis=1)
    bqkv = jnp.concatenate([bq, bk, bv])[None, :]
    qkv = _ln_qkv(x2, ln1_g[None, :], ln1_b[None, :], wqkv, bqkv)

    ctx2 = _attention(qkv)

    a, h2 = _proj_ln(ctx2, Wo, bo[None, :], x2, ln2_g[None, :], ln2_b[None, :])

    w1c = jnp.concatenate([sW1[0], sW1[1]], axis=1).astype(jnp.bfloat16)
    b1c = jnp.concatenate([sb1[0], sb1[1]])[None, :]
    w2c = jnp.concatenate([sW2[0], sW2[1]], axis=0).astype(jnp.bfloat16)
    b2s = (sb2[0] + sb2[1])[None, :]
    wr_p = jnp.zeros((H, NRP), jnp.float32).at[:, :NR].set(Wr)
    br_p = jnp.full((NRP,), -1e30, jnp.float32).at[:NR].set(br)[None, :]
    partial, tvp, tip = _shared_router(h2, a, w1c, b1c, w2c, b2s, wr_p, br_p)

    dst, tile_expert = _route_indices(tip[:, :TOPK])
    dst2 = dst.reshape(S, TOPK)
    xg = _sc_dispatch(h2, dst2[:, 0], dst2[:, 1])
    y_pad = _moe_ffn(tile_expert, xg, rW1.astype(jnp.bfloat16), rb1,
                     rW2.astype(jnp.bfloat16), rb2)
    yp = _sc_gather(y_pad, dst)
    out = _combine(partial, tvp, yp.reshape(S, TOPK * H))
    return out[None]


# speedup vs baseline: 1.5758x; 1.1607x over previous
import functools
import numpy as np
import jax
import jax.numpy as jnp
from jax.experimental import pallas as pl
from jax.experimental.pallas import tpu as pltpu
from jax.experimental.pallas import tpu_sc as plsc

S = 2048
H = 768
NH, HD = 12, 64
NR = 14
NS = 2
TOPK = 2
INTER = 768
NRW = 16
RT = 256
AT = 512
TILE = 256
SCBLK = 128
NPAIR = S * TOPK
PBUF = NPAIR + NR * TILE
NBLK = PBUF // TILE
SCALE = 1.0 / np.sqrt(HD)

try:
    _info = plsc.get_sparse_core_info()
    _NC, _NSUB = _info.num_cores, _info.num_subcores
except Exception:
    _NC, _NSUB = 2, 16
NW = _NC * _NSUB


def _gelu(x):
    return 0.5 * x * (1.0 + jax.lax.erf(x * np.float32(1.0 / np.sqrt(2.0))))


def _ln(x, g, b):
    m = jnp.mean(x, axis=-1, keepdims=True)
    v = jnp.mean((x - m) ** 2, axis=-1, keepdims=True)
    return (x - m) * jax.lax.rsqrt(v + 1e-5) * g + b


def _ln_qkv_body(x_ref, g_ref, b_ref, wq_ref, wk_ref, wv_ref,
                 bq_ref, bk_ref, bv_ref, o_ref):
    h = _ln(x_ref[...], g_ref[...], b_ref[...]).astype(jnp.bfloat16)
    o_ref[:, 0 * H:1 * H] = jnp.dot(h, wq_ref[...].astype(jnp.bfloat16),
                                    preferred_element_type=jnp.float32) + bq_ref[...]
    o_ref[:, 1 * H:2 * H] = jnp.dot(h, wk_ref[...].astype(jnp.bfloat16),
                                    preferred_element_type=jnp.float32) + bk_ref[...]
    o_ref[:, 2 * H:3 * H] = jnp.dot(h, wv_ref[...].astype(jnp.bfloat16),
                                    preferred_element_type=jnp.float32) + bv_ref[...]


def _attn_body(q_ref, k_ref, v_ref, o_ref):
    q = q_ref[...].astype(jnp.bfloat16)
    k = k_ref[...].astype(jnp.bfloat16)
    v = v_ref[...].astype(jnp.bfloat16)
    for h in range(NH):
        sl = slice(h * HD, (h + 1) * HD)
        s = jax.lax.dot_general(q[:, sl], k[:, sl], (((1,), (1,)), ((), ())),
                                preferred_element_type=jnp.float32) * SCALE
        m = jnp.max(s, axis=-1, keepdims=True)
        e = jnp.exp(s - m)
        sm = jnp.sum(e, axis=-1, keepdims=True)
        ctx = jnp.dot(e.astype(jnp.bfloat16), v[:, sl],
                      preferred_element_type=jnp.float32)
        o_ref[:, sl] = ctx * (1.0 / sm)


def _proj_moe_body(c_ref, w_ref, b_ref, x_ref, g_ref, bb_ref,
                   w1_ref, b1_ref, w2_ref, b2_ref, wr_ref, br_ref,
                   h2_ref, part_ref, tv_ref, ti_ref):
    a = jnp.dot(c_ref[...].astype(jnp.bfloat16), w_ref[...].astype(jnp.bfloat16),
                preferred_element_type=jnp.float32)
    a = a + b_ref[...] + x_ref[...]
    h = _ln(a, g_ref[...], bb_ref[...])
    h2_ref[...] = h
    hb = h.astype(jnp.bfloat16)
    shared = a + h + b2_ref[0:1, :] + b2_ref[1:2, :]
    for e in range(NS):
        act = _gelu(jnp.dot(hb, w1_ref[e].astype(jnp.bfloat16),
                            preferred_element_type=jnp.float32) + b1_ref[e:e + 1, :])
        shared += jnp.dot(act.astype(jnp.bfloat16), w2_ref[e].astype(jnp.bfloat16),
                          preferred_element_type=jnp.float32)
    part_ref[...] = shared
    logits = jnp.dot(h, wr_ref[...], preferred_element_type=jnp.float32) + br_ref[...]
    lm = jnp.max(logits, axis=-1, keepdims=True)
    ex = jnp.exp(logits - lm)
    aff = ex / jnp.sum(ex, axis=-1, keepdims=True)
    col = jax.lax.broadcasted_iota(jnp.int32, aff.shape, 1)
    i1 = jnp.argmax(aff, axis=-1)
    m1 = jnp.max(aff, axis=-1)
    masked = jnp.where(col == i1[:, None], -1.0, aff)
    i2 = jnp.argmax(masked, axis=-1)
    m2 = jnp.max(masked, axis=-1)
    colw = jax.lax.broadcasted_iota(jnp.int32, (aff.shape[0], NRW), 1)
    tv_ref[...] = jnp.where(colw == 0, m1[:, None],
                            jnp.where(colw == 1, m2[:, None], 0.0))
    ti_ref[...] = jnp.where(colw == 0, i1[:, None].astype(jnp.int32),
                            jnp.where(colw == 1, i2[:, None].astype(jnp.int32), 0))


def _moe_ffn_body(se_ref, xg_ref, w1_ref, b1_ref, w2_ref, b2_ref, o_ref):
    del se_ref
    act = _gelu(jnp.dot(xg_ref[...].astype(jnp.bfloat16),
                        w1_ref[0].astype(jnp.bfloat16),
                        preferred_element_type=jnp.float32) + b1_ref[0])
    o_ref[...] = jnp.dot(act.astype(jnp.bfloat16), w2_ref[0].astype(jnp.bfloat16),
                         preferred_element_type=jnp.float32) + b2_ref[0]


def _combine_body(p_ref, tv_ref, y_ref, o_ref):
    y = y_ref[...]
    tv = tv_ref[...]
    o_ref[...] = (p_ref[...] + tv[:, 0:1] * y[:, :H] + tv[:, 1:2] * y[:, H:])


def _sc_dispatch(h2, dst_even, dst_odd):
    rows_per_w = S // NW
    mesh = plsc.VectorSubcoreMesh(core_axis_name="c", subcore_axis_name="s")

    @functools.partial(
        pl.kernel, mesh=mesh,
        out_type=jax.ShapeDtypeStruct((PBUF, H), jnp.float32),
        scratch_types=[
            pltpu.VMEM((rows_per_w,), jnp.int32),
            pltpu.VMEM((rows_per_w,), jnp.int32),
            pltpu.VMEM((rows_per_w, H), jnp.float32),
            pltpu.SemaphoreType.DMA,
        ],
    )
    def k(h2_hbm, de_hbm, do_hbm, out_hbm, ie_v, io_v, rows_v, sem):
        wid = jax.lax.axis_index("s") * _NC + jax.lax.axis_index("c")
        base = wid * rows_per_w
        pltpu.sync_copy(de_hbm.at[pl.ds(base, rows_per_w)], ie_v)
        pltpu.sync_copy(do_hbm.at[pl.ds(base, rows_per_w)], io_v)
        pltpu.sync_copy(h2_hbm.at[pl.ds(base, rows_per_w)], rows_v)
        pltpu.async_copy(rows_v, out_hbm.at[ie_v], sem).wait()
        pltpu.async_copy(rows_v, out_hbm.at[io_v], sem).wait()

    return k(h2, dst_even, dst_odd)


def _sc_gather(table, idx):
    n = idx.shape[0]
    nblk = n // SCBLK
    mesh = plsc.VectorSubcoreMesh(core_axis_name="c", subcore_axis_name="s")

    @functools.partial(
        pl.kernel, mesh=mesh,
        out_type=jax.ShapeDtypeStruct((n, H), jnp.float32),
        scratch_types=[
            pltpu.VMEM((SCBLK,), jnp.int32),
            pltpu.VMEM((SCBLK, H), jnp.float32),
            pltpu.SemaphoreType.DMA,
        ],
    )
    def k(table_hbm, idx_hbm, out_hbm, idx_v, rows_v, sem):
        wid = jax.lax.axis_index("s") * _NC + jax.lax.axis_index("c")
        for j in range((nblk + NW - 1) // NW):
            t = wid + j * NW

            @pl.when(t < nblk)
            def _do():
                base = t * SCBLK
                pltpu.sync_copy(idx_hbm.at[pl.ds(base, SCBLK)], idx_v)
                pltpu.async_copy(table_hbm.at[idx_v], rows_v, sem).wait()
                pltpu.sync_copy(rows_v, out_hbm.at[pl.ds(base, SCBLK)])

    return k(table, idx)


def _ln_qkv(x2, g, b, wq, wk, wv, bq, bk, bv):
    wspec = pl.BlockSpec((H, H), lambda i: (0, 0))
    bspec = pl.BlockSpec((1, H), lambda i: (0, 0))
    return pl.pallas_call(
        _ln_qkv_body,
        grid=(S // RT,),
        in_specs=[
            pl.BlockSpec((RT, H), lambda i: (i, 0)),
            bspec, bspec, wspec, wspec, wspec, bspec, bspec, bspec,
        ],
        out_specs=pl.BlockSpec((RT, 3 * H), lambda i: (i, 0)),
        out_shape=jax.ShapeDtypeStruct((S, 3 * H), jnp.float32),
    )(x2, g, b, wq, wk, wv, bq, bk, bv)


def _attention(qkv):
    return pl.pallas_call(
        _attn_body,
        grid=(S // AT,),
        in_specs=[
            pl.BlockSpec((AT, H), lambda r: (r, 0)),
            pl.BlockSpec((S, H), lambda r: (0, 1)),
            pl.BlockSpec((S, H), lambda r: (0, 2)),
        ],
        out_specs=pl.BlockSpec((AT, H), lambda r: (r, 0)),
        out_shape=jax.ShapeDtypeStruct((S, H), jnp.float32),
    )(qkv, qkv, qkv)


def _proj_moe(ctx, wo, bo, x2, g2, b2, w1c, b1c, w2c, b2s, wr, br):
    return pl.pallas_call(
        _proj_moe_body,
        grid=(S // RT,),
        in_specs=[
            pl.BlockSpec((RT, H), lambda i: (i, 0)),
            pl.BlockSpec((H, H), lambda i: (0, 0)),
            pl.BlockSpec((1, H), lambda i: (0, 0)),
            pl.BlockSpec((RT, H), lambda i: (i, 0)),
            pl.BlockSpec((1, H), lambda i: (0, 0)),
            pl.BlockSpec((1, H), lambda i: (0, 0)),
            pl.BlockSpec((NS, H, INTER), lambda i: (0, 0, 0)),
            pl.BlockSpec((NS, INTER), lambda i: (0, 0)),
            pl.BlockSpec((NS, INTER, H), lambda i: (0, 0, 0)),
            pl.BlockSpec((NS, H), lambda i: (0, 0)),
            pl.BlockSpec((H, NR), lambda i: (0, 0)),
            pl.BlockSpec((1, NR), lambda i: (0, 0)),
        ],
        out_specs=[
            pl.BlockSpec((RT, H), lambda i: (i, 0)),
            pl.BlockSpec((RT, H), lambda i: (i, 0)),
            pl.BlockSpec((RT, NRW), lambda i: (i, 0)),
            pl.BlockSpec((RT, NRW), lambda i: (i, 0)),
        ],
        out_shape=[
            jax.ShapeDtypeStruct((S, H), jnp.float32),
            jax.ShapeDtypeStruct((S, H), jnp.float32),
            jax.ShapeDtypeStruct((S, NRW), jnp.float32),
            jax.ShapeDtypeStruct((S, NRW), jnp.int32),
        ],
    )(ctx, wo, bo, x2, g2, b2, w1c, b1c, w2c, b2s, wr, br)


def _moe_ffn(tile_expert, xg, rW1, rb1, rW2, rb2):
    grid_spec = pltpu.PrefetchScalarGridSpec(
        num_scalar_prefetch=1,
        grid=(NBLK,),
        in_specs=[
            pl.BlockSpec((TILE, H), lambda t, se: (t, 0)),
            pl.BlockSpec((1, H, INTER), lambda t, se: (se[t], 0, 0)),
            pl.BlockSpec((1, 1, INTER), lambda t, se: (se[t], 0, 0)),
            pl.BlockSpec((1, INTER, H), lambda t, se: (se[t], 0, 0)),
            pl.BlockSpec((1, 1, H), lambda t, se: (se[t], 0, 0)),
        ],
        out_specs=pl.BlockSpec((TILE, H), lambda t, se: (t, 0)),
    )
    return pl.pallas_call(
        _moe_ffn_body,
        grid_spec=grid_spec,
        out_shape=jax.ShapeDtypeStruct((PBUF, H), jnp.float32),
    )(tile_expert, xg, rW1, rb1[:, None, :], rW2, rb2[:, None, :])


def _combine(partial, tvp, yp2):
    return pl.pallas_call(
        _combine_body,
        grid=(S // RT,),
        in_specs=[
            pl.BlockSpec((RT, H), lambda i: (i, 0)),
            pl.BlockSpec((RT, NRW), lambda i: (i, 0)),
            pl.BlockSpec((RT, 2 * H), lambda i: (i, 0)),
        ],
        out_specs=pl.BlockSpec((RT, H), lambda i: (i, 0)),
        out_shape=jax.ShapeDtypeStruct((S, H), jnp.float32),
    )(partial, tvp, yp2)


def _route_indices(ti):
    e_p = ti.reshape(NPAIR)
    oh = (e_p[:, None] == jnp.arange(NR, dtype=jnp.int32)[None, :]).astype(jnp.int32)
    pc = jnp.cumsum(oh, axis=0)
    rank = jnp.take_along_axis(pc, e_p[:, None], axis=1)[:, 0] - 1
    counts = pc[-1]
    tiles_per = (counts + TILE - 1) // TILE
    ends = jnp.cumsum(tiles_per)
    base = (jnp.concatenate([jnp.zeros((1,), ends.dtype), ends[:-1]]) * TILE).astype(jnp.int32)
    dst = base[e_p] + rank
    tile_expert = jnp.minimum(
        jnp.searchsorted(ends, jnp.arange(NBLK, dtype=ends.dtype), side="right"),
        NR - 1).astype(jnp.int32)
    return dst, tile_expert


def kernel(x, ln1_g, ln1_b, ln2_g, ln2_b, Wq, bq, Wk, bk, Wv, bv, Wo, bo,
           Wr, br, sW1, sb1, sW2, sb2, rW1, rb1, rW2, rb2):
    x2 = x[0]

    qkv = _ln_qkv(x2, ln1_g[None, :], ln1_b[None, :], Wq, Wk, Wv,
                  bq[None, :], bk[None, :], bv[None, :])

    ctx2 = _attention(qkv)

    h2, partial, tvp, tip = _proj_moe(
        ctx2, Wo, bo[None, :], x2, ln2_g[None, :], ln2_b[None, :],
        sW1, sb1, sW2, sb2, Wr, br[None, :])

    dst, tile_expert = _route_indices(tip[:, :TOPK])
    dst2 = dst.reshape(S, TOPK)
    xg = _sc_dispatch(h2, dst2[:, 0], dst2[:, 1])
    y_pad = _moe_ffn(tile_expert, xg, rW1, rb1, rW2, rb2)
    yp = _sc_gather(y_pad, dst)
    out = _combine(partial, tvp, yp.reshape(S, TOPK * H))
    return out[None]
